# Initial kernel scaffold; baseline (speedup 1.0000x reference)
import functools

import jax
import jax.numpy as jnp
from jax import lax
from jax.experimental import pallas as pl
from jax.experimental.pallas import tpu as pltpu
from jax.experimental.pallas import tpu_sc as plsc

_T = 8192
_E = 256
_G = 8
_EPG = 32
_K = 8
_KG = 4
_SCALE = 2.5

_NW = 32
_TPW = _T // _NW
_PAIRS = _TPW // 2
_WORDS = _TPW * _E


def _body(logits_hbm, bias_hbm, w_hbm, id_hbm, buf, bias_v, w_out, id_out,
          sel, stage_w, stage_i):
  c = lax.axis_index("c")
  s = lax.axis_index("s")
  wid = s * 2 + c

  pltpu.sync_copy(bias_hbm, bias_v)
  pltpu.sync_copy(logits_hbm.at[pl.ds(wid * _WORDS, _WORDS)], buf)

  iota = lax.iota(jnp.int32, 16)
  lane_lt8 = iota < 8
  trans_base = iota * _EPG
  neg_inf = jnp.float32(-jnp.inf)

  def pair_body(pair, carry):
    base = pl.multiple_of(pair * (2 * _E), 2 * _E)

    for k in range(2 * _E // 16):
      x = buf[pl.ds(base + 16 * k, 16)]
      b = bias_v[pl.ds((k % 16) * 16, 16)]
      s4c = jnp.float32(1.0) / (jnp.float32(1.0) + jnp.exp(-x)) + b
      buf[pl.ds(base + 16 * k, 16)] = s4c

    m1 = jnp.full((16,), neg_inf, jnp.float32)
    m2 = jnp.full((16,), neg_inf, jnp.float32)
    gidx0 = trans_base + base
    for j in range(_EPG):
      x = plsc.load_gather(buf, [gidx0 + j])
      m2 = jnp.maximum(m2, jnp.minimum(m1, x))
      m1 = jnp.maximum(m1, x)
    gs = m1 + m2

    gs0 = jnp.where(lane_lt8, gs, neg_inf)
    _, v0 = plsc.sort_key_val(gs0, iota, descending=True)
    gs1r = jnp.where(lane_lt8, lax.rev(gs, (0,)), neg_inf)
    _, v1 = plsc.sort_key_val(gs1r, 7 - iota, descending=True)
    sel[pl.ds(0, 16)] = v0
    sel[pl.ds(16, 16)] = v1

    for tok in range(2):
      tok_base = base + tok * _E
      cur_k = jnp.full((16,), neg_inf, jnp.float32)
      cur_v = jnp.zeros((16,), jnp.int32)
      for gi in range(_KG):
        g = sel[16 * tok + gi]
        for h in range(2):
          off = tok_base + g * _EPG + h * 16
          cand_k = buf[pl.ds(off, 16)]
          cand_v = g * _EPG + h * 16 + iota
          cand_k, cand_v = plsc.sort_key_val(cand_k, cand_v,
                                             descending=True)
          rk = lax.rev(cand_k, (0,))
          rv = lax.rev(cand_v, (0,))
          keep = cur_k >= rk
          hi_k = jnp.where(keep, cur_k, rk)
          hi_v = jnp.where(keep, cur_v, rv)
          cur_k, cur_v = plsc.sort_key_val(hi_k, hi_v, descending=True)

      s4c_g = plsc.load_gather(buf, [tok_base + cur_v])
      bias_g = plsc.load_gather(bias_v, [cur_v])
      w = s4c_g - bias_g
      wm = jnp.where(lane_lt8, w, jnp.float32(0.0))
      tot = jnp.sum(wm)
      wfin = wm / (tot + jnp.float32(1e-20)) * jnp.float32(_SCALE)
      stage_w[pl.ds(tok * 8, 16)] = wfin
      stage_i[pl.ds(tok * 8, 16)] = cur_v

    w_out[pl.ds(pair * 16, 16)] = stage_w[pl.ds(0, 16)]
    id_out[pl.ds(pair * 16, 16)] = stage_i[pl.ds(0, 16)]
    return carry

  lax.fori_loop(0, _PAIRS, pair_body, 0)

  out_base = wid * (_TPW * _K)
  pltpu.sync_copy(w_out, w_hbm.at[pl.ds(out_base, _TPW * _K)])
  pltpu.sync_copy(id_out, id_hbm.at[pl.ds(out_base, _TPW * _K)])


@jax.jit
def _run(logits_flat, bias):
  mesh = plsc.VectorSubcoreMesh(core_axis_name="c", subcore_axis_name="s")
  kfn = pl.kernel(
      _body,
      out_type=(
          jax.ShapeDtypeStruct((_T * _K,), jnp.float32),
          jax.ShapeDtypeStruct((_T * _K,), jnp.int32),
      ),
      mesh=mesh,
      scratch_types=[
          pltpu.VMEM((_WORDS,), jnp.float32),
          pltpu.VMEM((_E,), jnp.float32),
          pltpu.VMEM((_TPW * _K,), jnp.float32),
          pltpu.VMEM((_TPW * _K,), jnp.int32),
          pltpu.VMEM((32,), jnp.int32),
          pltpu.VMEM((32,), jnp.float32),
          pltpu.VMEM((32,), jnp.int32),
      ],
  )
  return kfn(logits_flat, bias)


def kernel(router_logits, e_score_correction_bias):
  w, ids = _run(router_logits.reshape(-1).astype(jnp.float32),
                e_score_correction_bias.astype(jnp.float32))
  return w.reshape(_T, _K), ids.reshape(_T, _K)


# speedup vs baseline: 29.4602x; 29.4602x over previous
import functools

import jax
import jax.numpy as jnp
from jax import lax
from jax.experimental import pallas as pl
from jax.experimental.pallas import tpu as pltpu
from jax.experimental.pallas import tpu_sc as plsc

_T = 8192
_E = 256
_G = 8
_EPG = 32
_K = 8
_KG = 4
_SCALE = 2.5

_NW = 32
_TPW = _T // _NW
_PAIRS = _TPW // 2
_WORDS = _TPW * _E


def _body(logits_hbm, bias_hbm, w_hbm, id_hbm, buf, bias_v, w_out, id_out,
          stage_w, stage_i):
  c = lax.axis_index("c")
  s = lax.axis_index("s")
  wid = s * 2 + c

  pltpu.sync_copy(bias_hbm, bias_v)
  pltpu.sync_copy(logits_hbm.at[pl.ds(wid * _WORDS, _WORDS)], buf)

  iota = lax.iota(jnp.int32, 16)
  lane_lt8 = iota < 8
  trans_base = iota * _EPG
  neg_inf = jnp.float32(-jnp.inf)

  def pair_body(pair, carry):
    base = pl.multiple_of(pair * (2 * _E), 2 * _E)

    for k in range(2 * _E // 16):
      x = buf[pl.ds(base + 16 * k, 16)]
      b = bias_v[pl.ds((k % 16) * 16, 16)]
      s4c = jnp.float32(1.0) / (jnp.float32(1.0) + jnp.exp(-x)) + b
      buf[pl.ds(base + 16 * k, 16)] = s4c

    m1 = jnp.full((16,), neg_inf, jnp.float32)
    m2 = jnp.full((16,), neg_inf, jnp.float32)
    gidx0 = trans_base + base
    for j in range(_EPG):
      x = plsc.load_gather(buf, [gidx0 + j])
      m2 = jnp.maximum(m2, jnp.minimum(m1, x))
      m1 = jnp.maximum(m1, x)
    gs = m1 + m2

    gs0 = jnp.where(lane_lt8, gs, neg_inf)
    _, v0 = plsc.sort_key_val(gs0, iota, descending=True)
    gs1r = jnp.where(lane_lt8, lax.rev(gs, (0,)), neg_inf)
    _, v1 = plsc.sort_key_val(gs1r, 7 - iota, descending=True)
    sel_vals = (v0, v1)

    for tok in range(2):
      tok_base = base + tok * _E
      cur_k = jnp.full((16,), neg_inf, jnp.float32)
      cur_v = jnp.zeros((16,), jnp.int32)
      for gi in range(_KG):
        g = sel_vals[tok][gi]
        for h in range(2):
          off = tok_base + g * _EPG + h * 16
          cand_k = buf[pl.ds(off, 16)]
          cand_v = g * _EPG + h * 16 + iota
          cand_k, cand_v = plsc.sort_key_val(cand_k, cand_v,
                                             descending=True)
          rk = lax.rev(cand_k, (0,))
          rv = lax.rev(cand_v, (0,))
          keep = cur_k >= rk
          hi_k = jnp.where(keep, cur_k, rk)
          hi_v = jnp.where(keep, cur_v, rv)
          cur_k, cur_v = plsc.sort_key_val(hi_k, hi_v, descending=True)

      s4c_g = plsc.load_gather(buf, [tok_base + cur_v])
      bias_g = plsc.load_gather(bias_v, [cur_v])
      w = s4c_g - bias_g
      wm = jnp.where(lane_lt8, w, jnp.float32(0.0))
      tot = jnp.sum(wm)
      wfin = wm / (tot + jnp.float32(1e-20)) * jnp.float32(_SCALE)
      stage_w[pl.ds(tok * 8, 16)] = wfin
      stage_i[pl.ds(tok * 8, 16)] = cur_v

    w_out[pl.ds(pair * 16, 16)] = stage_w[pl.ds(0, 16)]
    id_out[pl.ds(pair * 16, 16)] = stage_i[pl.ds(0, 16)]
    return carry

  lax.fori_loop(0, _PAIRS, pair_body, 0)

  out_base = wid * (_TPW * _K)
  pltpu.sync_copy(w_out, w_hbm.at[pl.ds(out_base, _TPW * _K)])
  pltpu.sync_copy(id_out, id_hbm.at[pl.ds(out_base, _TPW * _K)])


@jax.jit
def _run(logits_flat, bias):
  mesh = plsc.VectorSubcoreMesh(core_axis_name="c", subcore_axis_name="s")
  kfn = pl.kernel(
      _body,
      out_type=(
          jax.ShapeDtypeStruct((_T * _K,), jnp.float32),
          jax.ShapeDtypeStruct((_T * _K,), jnp.int32),
      ),
      mesh=mesh,
      compiler_params=pltpu.CompilerParams(needs_layout_passes=False),
      scratch_types=[
          pltpu.VMEM((_WORDS,), jnp.float32),
          pltpu.VMEM((_E,), jnp.float32),
          pltpu.VMEM((_TPW * _K,), jnp.float32),
          pltpu.VMEM((_TPW * _K,), jnp.int32),
          pltpu.VMEM((32,), jnp.float32),
          pltpu.VMEM((32,), jnp.int32),
      ],
  )
  return kfn(logits_flat, bias)


def kernel(router_logits, e_score_correction_bias):
  w, ids = _run(router_logits.reshape(-1).astype(jnp.float32),
                e_score_correction_bias.astype(jnp.float32))
  return w.reshape(_T, _K), ids.reshape(_T, _K)


# speedup vs baseline: 56.2312x; 1.9087x over previous
import jax
import jax.numpy as jnp
from jax import lax
from jax.experimental import pallas as pl
from jax.experimental.pallas import tpu as pltpu
from jax.experimental.pallas import tpu_sc as plsc

_T = 8192
_E = 256
_G = 8
_EPG = 32
_K = 8
_KG = 4
_SCALE = 2.5

_NW = 32
_TPW = _T // _NW
_PAIRS = _TPW // 2
_WORDS = _TPW * _E

_TC_BLK = 1024


def _tc_body(x_ref, b_ref, o_ref):
  x = x_ref[...]
  o_ref[...] = jnp.float32(1.0) / (jnp.float32(1.0) + jnp.exp(-x)) + b_ref[...]


def _rcp(d):
  r = plsc.bitcast(jnp.int32(0x7EF311C7) - plsc.bitcast(d, jnp.int32),
                   jnp.float32)
  for _ in range(3):
    r = r * (jnp.float32(2.0) - d * r)
  return r


def _merge_top16(ak, av, bk, bv):
  rk = lax.rev(bk, (0,))
  rv = lax.rev(bv, (0,))
  keep = ak >= rk
  hi_k = jnp.where(keep, ak, rk)
  hi_v = jnp.where(keep, av, rv)
  return plsc.sort_key_val(hi_k, hi_v, descending=True)


def _sc_body(s4c_hbm, bias_hbm, w_hbm, id_hbm, buf, bias_v, w_out, id_out):
  c = lax.axis_index("c")
  s = lax.axis_index("s")
  wid = s * 2 + c

  pltpu.sync_copy(bias_hbm, bias_v)
  pltpu.sync_copy(s4c_hbm.at[pl.ds(wid * _WORDS, _WORDS)], buf)

  iota = lax.iota(jnp.int32, 16)
  lane_lt8 = iota < 8
  trans_base = iota * _EPG
  neg_inf = jnp.float32(-jnp.inf)

  @plsc.parallel_loop(0, _PAIRS, unroll=2)
  def pair_body(pair):
    base = pl.multiple_of(pair * (2 * _E), 2 * _E)

    gidx0 = trans_base + base
    m1s, m2s = [], []
    for chunk in range(4):
      m1 = jnp.full((16,), neg_inf, jnp.float32)
      m2 = jnp.full((16,), neg_inf, jnp.float32)
      for j in range(chunk * 8, chunk * 8 + 8):
        x = plsc.load_gather(buf, [gidx0 + j])
        m2 = jnp.maximum(m2, jnp.minimum(m1, x))
        m1 = jnp.maximum(m1, x)
      m1s.append(m1)
      m2s.append(m2)

    def comb(a1, a2, b1, b2):
      return (jnp.maximum(a1, b1),
              jnp.maximum(jnp.minimum(a1, b1), jnp.maximum(a2, b2)))

    x1, x2 = comb(m1s[0], m2s[0], m1s[1], m2s[1])
    y1, y2 = comb(m1s[2], m2s[2], m1s[3], m2s[3])
    g1, g2 = comb(x1, x2, y1, y2)
    gs = g1 + g2

    gs0 = jnp.where(lane_lt8, gs, neg_inf)
    _, v0 = plsc.sort_key_val(gs0, iota, descending=True)
    gs1r = jnp.where(lane_lt8, lax.rev(gs, (0,)), neg_inf)
    _, v1 = plsc.sort_key_val(gs1r, 7 - iota, descending=True)
    sel_vals = (v0, v1)

    for tok in range(2):
      tok_base = base + tok * _E
      srt = []
      for gi in range(_KG):
        g = sel_vals[tok][gi]
        for h in range(2):
          off = tok_base + g * _EPG + h * 16
          cand_k = buf[pl.ds(off, 16)]
          cand_v = g * _EPG + h * 16 + iota
          srt.append(plsc.sort_key_val(cand_k, cand_v, descending=True))
      lvl = srt
      while len(lvl) > 1:
        nxt = []
        for i in range(0, len(lvl), 2):
          nxt.append(_merge_top16(lvl[i][0], lvl[i][1],
                                  lvl[i + 1][0], lvl[i + 1][1]))
        lvl = nxt
      cur_k, cur_v = lvl[0]

      s4c_g = plsc.load_gather(buf, [tok_base + cur_v])
      bias_g = plsc.load_gather(bias_v, [cur_v])
      w = s4c_g - bias_g
      wm = jnp.where(lane_lt8, w, jnp.float32(0.0))
      tot = jnp.sum(wm)
      rn = _rcp(jnp.broadcast_to(tot, (16,)) + jnp.float32(1e-20))
      wfin = wm * rn * jnp.float32(_SCALE)
      out_off = pair * 16 + tok * 8
      plsc.store_compressed(w_out.at[pl.ds(out_off, 16)], wfin,
                            mask=lane_lt8)
      plsc.store_compressed(id_out.at[pl.ds(out_off, 16)], cur_v,
                            mask=lane_lt8)

  out_base = wid * (_TPW * _K)
  pltpu.sync_copy(w_out.at[pl.ds(0, _TPW * _K)],
                  w_hbm.at[pl.ds(out_base, _TPW * _K)])
  pltpu.sync_copy(id_out.at[pl.ds(0, _TPW * _K)],
                  id_hbm.at[pl.ds(out_base, _TPW * _K)])


@jax.jit
def _run(logits, bias):
  s4c = pl.pallas_call(
      _tc_body,
      grid=(_T // _TC_BLK,),
      in_specs=[
          pl.BlockSpec((_TC_BLK, _E), lambda i: (i, 0)),
          pl.BlockSpec((1, _E), lambda i: (0, 0)),
      ],
      out_specs=pl.BlockSpec((_TC_BLK, _E), lambda i: (i, 0)),
      out_shape=jax.ShapeDtypeStruct((_T, _E), jnp.float32),
  )(logits, bias.reshape(1, _E))

  mesh = plsc.VectorSubcoreMesh(core_axis_name="c", subcore_axis_name="s")
  kfn = pl.kernel(
      _sc_body,
      out_type=(
          jax.ShapeDtypeStruct((_T * _K,), jnp.float32),
          jax.ShapeDtypeStruct((_T * _K,), jnp.int32),
      ),
      mesh=mesh,
      compiler_params=pltpu.CompilerParams(needs_layout_passes=False),
      scratch_types=[
          pltpu.VMEM((_WORDS,), jnp.float32),
          pltpu.VMEM((_E,), jnp.float32),
          pltpu.VMEM((_TPW * _K + 16,), jnp.float32),
          pltpu.VMEM((_TPW * _K + 16,), jnp.int32),
      ],
  )
  return kfn(s4c.reshape(-1), bias)


def kernel(router_logits, e_score_correction_bias):
  w, ids = _run(router_logits.astype(jnp.float32),
                e_score_correction_bias.astype(jnp.float32))
  return w.reshape(_T, _K), ids.reshape(_T, _K)
